# baseline (speedup 1.0000x reference)
import functools

import jax
import jax.numpy as jnp
from jax import lax
from jax.experimental import pallas as pl
from jax.experimental.pallas import tpu as pltpu
from jax.experimental.pallas import tpu_sc as plsc

D = 128
VPAD = 104
NC, NS = 2, 16
NW = NC * NS
CHUNK = 48
NBUF = 3


def _table_body(nuc_ref, elec_ref, we_ref, wf_ref, b_ref, out_ref):
    v = nuc_ref.shape[0]
    comb = nuc_ref[...] + jnp.dot(
        elec_ref[...], we_ref[...], preferred_element_type=jnp.float32
    )
    h = jnp.dot(comb, wf_ref[...], preferred_element_type=jnp.float32) + b_ref[...]
    t = h * jax.nn.sigmoid(h)
    out_ref[...] = jnp.concatenate(
        [t, jnp.zeros((VPAD - v, D), jnp.float32)], axis=0
    )


def _fused_table(nuclear_table, elec_features, W_elec, W_final, b_final):
    return pl.pallas_call(
        _table_body,
        out_shape=jax.ShapeDtypeStruct((VPAD, D), jnp.float32),
    )(nuclear_table, elec_features, W_elec, W_final, b_final.reshape(1, D))


def _make_gather(n):
    assert n % 8 == 0 and n >= CHUNK
    nchunks_total = -(-n // CHUNK)
    cpw = -(-nchunks_total // NW)
    stage = cpw * CHUNK
    mesh = plsc.VectorSubcoreMesh(core_axis_name="c", subcore_axis_name="s")

    @functools.partial(
        pl.kernel,
        mesh=mesh,
        out_type=jax.ShapeDtypeStruct((n, D), jnp.float32),
        scratch_types=[
            pltpu.VMEM((stage,), jnp.int32),
            pltpu.VMEM((NBUF, CHUNK, D), jnp.float32),
            pltpu.VMEM_SHARED((VPAD, D), jnp.float32),
        ]
        + [pltpu.SemaphoreType.DMA] * NBUF,
    )
    def gather_k(
        table_hbm, idx_hbm, out_hbm, idx_v, rows_v, table_spm,
        sem_a, sem_b, sem_c,
    ):
        sems = (sem_a, sem_b, sem_c)
        wid = lax.axis_index("s") * NC + lax.axis_index("c")

        @pl.when(lax.axis_index("s") == 0)
        def _():
            pltpu.sync_copy(table_hbm, table_spm)

        t0 = wid * cpw
        nch = jnp.clip(nchunks_total - t0, 0, cpw)
        stage_start = pl.multiple_of(jnp.minimum(t0 * CHUNK, n - stage), 8)
        pltpu.sync_copy(idx_hbm.at[pl.ds(stage_start, stage)], idx_v)
        plsc.subcore_barrier()

        def goff(c):
            return pl.multiple_of(jnp.minimum((t0 + c) * CHUNK, n - CHUNK), 8)

        def idx_slice(c):
            return idx_v.at[pl.ds(goff(c) - stage_start, CHUNK)]

        def start(c, slot, sem):
            pltpu.async_copy(table_spm.at[idx_slice(c)], rows_v.at[slot], sem)

        def finish(c, slot, sem):
            pltpu.make_async_copy(
                table_spm.at[idx_slice(c)], rows_v.at[slot], sem
            ).wait()
            pltpu.sync_copy(rows_v.at[slot], out_hbm.at[pl.ds(goff(c), CHUNK)])

        for b in range(NBUF):

            @pl.when(b < nch)
            def _(b=b):
                start(b, b, sems[b])

        def body(i, carry):
            for b in range(NBUF):
                c = i * NBUF + b

                if b == 0:
                    finish(c, 0, sems[0])
                else:

                    @pl.when(c < nch)
                    def _(b=b, c=c):
                        finish(c, b, sems[b])

                @pl.when(c + NBUF < nch)
                def _(b=b, c=c):
                    start(c + NBUF, b, sems[b])

            return carry

        lax.fori_loop(0, (nch + NBUF - 1) // NBUF, body, 0)

    return gather_k


def kernel(atom_types, elec_features, nuclear_table, W_elec, W_final, b_final):
    n = atom_types.shape[0]
    table = _fused_table(nuclear_table, elec_features, W_elec, W_final, b_final)
    return _make_gather(n)(table, atom_types.astype(jnp.int32))


# speedup vs baseline: 1.0418x; 1.0418x over previous
import functools

import jax
import jax.numpy as jnp
from jax import lax
from jax.experimental import pallas as pl
from jax.experimental.pallas import tpu as pltpu
from jax.experimental.pallas import tpu_sc as plsc

D = 128
VPAD = 104
NC, NS = 2, 16
NW = NC * NS
CHUNK = 64
NBUF = 3


def _table_body(nuc_ref, elec_ref, we_ref, wf_ref, b_ref, out_ref):
    v = nuc_ref.shape[0]
    comb = nuc_ref[...] + jnp.dot(
        elec_ref[...], we_ref[...], preferred_element_type=jnp.float32
    )
    h = jnp.dot(comb, wf_ref[...], preferred_element_type=jnp.float32) + b_ref[...]
    t = h * jax.nn.sigmoid(h)
    out_ref[...] = jnp.concatenate(
        [t, jnp.zeros((VPAD - v, D), jnp.float32)], axis=0
    )


def _fused_table(nuclear_table, elec_features, W_elec, W_final, b_final):
    return pl.pallas_call(
        _table_body,
        out_shape=jax.ShapeDtypeStruct((VPAD, D), jnp.float32),
    )(nuclear_table, elec_features, W_elec, W_final, b_final.reshape(1, D))


def _make_gather(n):
    assert n % 8 == 0 and n >= CHUNK
    nchunks_total = -(-n // CHUNK)
    cpw = -(-nchunks_total // NW)
    stage = cpw * CHUNK
    mesh = plsc.VectorSubcoreMesh(core_axis_name="c", subcore_axis_name="s")

    @functools.partial(
        pl.kernel,
        mesh=mesh,
        out_type=jax.ShapeDtypeStruct((n, D), jnp.float32),
        scratch_types=[
            pltpu.VMEM((stage,), jnp.int32),
            pltpu.VMEM((NBUF, CHUNK, D), jnp.float32),
            pltpu.VMEM_SHARED((VPAD, D), jnp.float32),
        ]
        + [pltpu.SemaphoreType.DMA] * NBUF,
    )
    def gather_k(
        table_hbm, idx_hbm, out_hbm, idx_v, rows_v, table_spm,
        sem_a, sem_b, sem_c,
    ):
        sems = (sem_a, sem_b, sem_c)
        wid = lax.axis_index("s") * NC + lax.axis_index("c")

        @pl.when(lax.axis_index("s") == 0)
        def _():
            pltpu.async_copy(table_hbm, table_spm, sem_a)

        t0 = wid * cpw
        nch = jnp.clip(nchunks_total - t0, 0, cpw)
        stage_start = pl.multiple_of(jnp.minimum(t0 * CHUNK, n - stage), 8)
        pltpu.sync_copy(idx_hbm.at[pl.ds(stage_start, stage)], idx_v)

        @pl.when(lax.axis_index("s") == 0)
        def _():
            pltpu.make_async_copy(table_hbm, table_spm, sem_a).wait()

        plsc.subcore_barrier()

        def goff(c):
            return pl.multiple_of(jnp.minimum((t0 + c) * CHUNK, n - CHUNK), 8)

        def idx_slice(c):
            return idx_v.at[pl.ds(goff(c) - stage_start, CHUNK)]

        def start(c, slot, sem):
            pltpu.async_copy(table_spm.at[idx_slice(c)], rows_v.at[slot], sem)

        def finish(c, slot, sem):
            pltpu.make_async_copy(
                table_spm.at[idx_slice(c)], rows_v.at[slot], sem
            ).wait()
            pltpu.sync_copy(rows_v.at[slot], out_hbm.at[pl.ds(goff(c), CHUNK)])

        for b in range(NBUF):

            @pl.when(b < nch)
            def _(b=b):
                start(b, b, sems[b])

        def body(i, carry):
            for b in range(NBUF):
                c = i * NBUF + b

                if b == 0:
                    finish(c, 0, sems[0])
                else:

                    @pl.when(c < nch)
                    def _(b=b, c=c):
                        finish(c, b, sems[b])

                @pl.when(c + NBUF < nch)
                def _(b=b, c=c):
                    start(c + NBUF, b, sems[b])

            return carry

        lax.fori_loop(0, (nch + NBUF - 1) // NBUF, body, 0)

    return gather_k


def kernel(atom_types, elec_features, nuclear_table, W_elec, W_final, b_final):
    n = atom_types.shape[0]
    table = _fused_table(nuclear_table, elec_features, W_elec, W_final, b_final)
    return _make_gather(n)(table, atom_types.astype(jnp.int32))
